# baseline (speedup 1.0000x reference)
import functools

import jax
import jax.numpy as jnp
from jax import lax
from jax.experimental import pallas as pl
from jax.experimental.pallas import tpu as pltpu
from jax.experimental.pallas import tpu_sc as plsc

NC = 2
NS = 16
DH = 16
EPS_BN_ = 1e-5


def _chunk(rows, cap):
    ch = min(rows, cap)
    while rows % ch:
        ch -= 1
    return ch


def _chunk8(total, cap):
    cand = [d for d in range(8, cap + 1, 8) if total % d == 0]
    assert cand, (total, cap)
    good = [d for d in cand if total // d >= NS]
    return max(good) if good else min(cand)


def _edge_phase(h128, src, dst, attr_flat, ew2, eb2, *, interpret=False):
    N = h128.shape[0]
    E = src.shape[0]
    ew = E // NS
    W = _chunk(ew, 80)
    assert W % DH == 0 and W % 8 == 0
    nwin = ew // W
    assert nwin % 2 == 0
    ch = _chunk8(N, W)
    nq = N // ch

    mesh = plsc.VectorSubcoreMesh(core_axis_name="c", subcore_axis_name="s",
                                  num_cores=NC, num_subcores=NS)

    def body(h_hbm, src_hbm, dst_hbm, attr_hbm, ew_hbm, eb_hbm,
             out0, out1, ew_v, eb_v,
             s_v0, s_v1, d_v0, d_v1, a_v0, a_v1, r_v0, r_v1, m_v0, m_v1,
             sd_v0, sd_v1, aggr_sh, lsem0, lsem1, gsem0, gsem1, csem0, csem1):
        c = lax.axis_index("c")
        s = lax.axis_index("s")
        s_v = [s_v0, s_v1]
        d_v = [d_v0, d_v1]
        a_v = [a_v0, a_v1]
        r_v = [r_v0, r_v1]
        m_v = [m_v0, m_v1]
        sd_v = [sd_v0, sd_v1]
        lsem = [lsem0, lsem1]
        gsem = [gsem0, gsem1]
        csem = [csem0, csem1]

        zero16 = jnp.zeros((DH,), jnp.float32)

        def zrow(i, carry):
            m_v0[i] = zero16
            return carry
        lax.fori_loop(0, ch, zrow, 0)
        nq_s = lax.div(jnp.int32(nq) - s + jnp.int32(NS) - 1, jnp.int32(NS))

        def zchunk(k, carry):
            off = pl.multiple_of((s + k * NS) * ch, 8)
            pltpu.sync_copy(m_v0.at[pl.ds(0, ch)],
                            aggr_sh.at[pl.ds(off, ch)])
            return carry
        lax.fori_loop(0, nq_s, zchunk, 0)
        plsc.subcore_barrier()

        for cc in range(NC):
            @pl.when(c == cc)
            def _():
                pltpu.sync_copy(ew_hbm.at[pl.ds(cc * 3 * DH, 3 * DH)], ew_v)
                pltpu.sync_copy(eb_hbm.at[pl.ds(cc * DH, DH)], eb_v)
        w0 = ew_v[pl.ds(0, DH)]
        w1 = ew_v[pl.ds(DH, DH)]
        w2 = ew_v[pl.ds(2 * DH, DH)]
        bb = eb_v[...]

        base0 = s * ew
        lane0 = c * DH

        def lin_descs(g, k):
            base = pl.multiple_of(base0 + g * W, 8)
            base3 = pl.multiple_of(base * 3, 8)
            return (
                pltpu.make_async_copy(src_hbm.at[pl.ds(base, W)], s_v[k],
                                      lsem[k]),
                pltpu.make_async_copy(dst_hbm.at[pl.ds(base, W)], d_v[k],
                                      lsem[k]),
                pltpu.make_async_copy(attr_hbm.at[pl.ds(base3, 3 * W)],
                                      a_v[k].at[pl.ds(0, 3 * W)], lsem[k]),
            )

        def issue_lin(g, k):
            for d in lin_descs(g, k):
                d.start()

        def wait_lin(g, k):
            for d in lin_descs(g, k):
                d.wait()

        def gat_desc(k):
            return pltpu.make_async_copy(h_hbm.at[s_v[k]], r_v[k], gsem[k])

        def sc_desc(k):
            return pltpu.make_async_copy(m_v[k], aggr_sh.at[sd_v[k]], csem[k])

        iota16 = lax.iota(jnp.int32, DH)

        def compute(k):
            for q in range(W // DH):
                jq = iota16 + (q * DH)
                a0 = plsc.load_gather(a_v[k], [jq * 3])
                a1 = plsc.load_gather(a_v[k], [jq * 3 + 1])
                a2 = plsc.load_gather(a_v[k], [jq * 3 + 2])
                for fi in range(DH):
                    lane = jnp.broadcast_to(lane0 + fi, (DH,)).astype(jnp.int32)
                    hf = plsc.load_gather(r_v[k], [jq, lane])
                    ef = a0 * w0[fi] + a1 * w1[fi] + a2 * w2[fi] + bb[fi]
                    m16 = jnp.maximum(hf + ef, 0.0)
                    fv = jnp.broadcast_to(jnp.int32(fi), (DH,))
                    plsc.store_scatter(m_v[k], [jq, fv], m16)
            for q in range(W // DH):
                sd_v[k][pl.ds(q * DH, DH)] = d_v[k][pl.ds(q * DH, DH)]

        def half(t, g, k):
            @pl.when(g + 1 < nwin)
            def _():
                wait_lin(g + 1, k ^ 1)
                gat_desc(k ^ 1).start()
            gat_desc(k).wait()
            @pl.when(g >= 2)
            def _():
                sc_desc(k).wait()
            compute(k)
            sc_desc(k).start(add=True)
            @pl.when(g + 2 < nwin)
            def _():
                issue_lin(g + 2, k)

        issue_lin(0, 0)
        issue_lin(1, 1)
        wait_lin(0, 0)
        gat_desc(0).start()

        def body2(t, carry):
            half(t, 2 * t, 0)
            half(t, 2 * t + 1, 1)
            return carry
        lax.fori_loop(0, nwin // 2, body2, 0)
        sc_desc(0).wait()
        sc_desc(1).wait()
        plsc.subcore_barrier()

        def fchunk(k, carry):
            off = pl.multiple_of((s + k * NS) * ch, 8)
            sl = pl.ds(off, ch)
            pltpu.sync_copy(aggr_sh.at[sl], m_v0.at[pl.ds(0, ch)])
            @pl.when(c == 0)
            def _():
                pltpu.sync_copy(m_v0.at[pl.ds(0, ch)], out0.at[sl])
            @pl.when(c == 1)
            def _():
                pltpu.sync_copy(m_v0.at[pl.ds(0, ch)], out1.at[sl])
            return carry
        lax.fori_loop(0, nq_s, fchunk, 0)

    f = pl.kernel(
        body,
        out_type=(jax.ShapeDtypeStruct((N, DH), jnp.float32),
                  jax.ShapeDtypeStruct((N, DH), jnp.float32)),
        mesh=mesh,
        scratch_types=[
            pltpu.VMEM((3 * DH,), jnp.float32),
            pltpu.VMEM((DH,), jnp.float32),
            pltpu.VMEM((W,), jnp.int32),
            pltpu.VMEM((W,), jnp.int32),
            pltpu.VMEM((W,), jnp.int32),
            pltpu.VMEM((W,), jnp.int32),
            pltpu.VMEM((3 * W + DH,), jnp.float32),
            pltpu.VMEM((3 * W + DH,), jnp.float32),
            pltpu.VMEM((W, 128), jnp.float32),
            pltpu.VMEM((W, 128), jnp.float32),
            pltpu.VMEM((W, DH), jnp.float32),
            pltpu.VMEM((W, DH), jnp.float32),
            pltpu.VMEM((W,), jnp.int32),
            pltpu.VMEM((W,), jnp.int32),
            pltpu.VMEM_SHARED((N, DH), jnp.float32),
            pltpu.SemaphoreType.DMA,
            pltpu.SemaphoreType.DMA,
            pltpu.SemaphoreType.DMA,
            pltpu.SemaphoreType.DMA,
            pltpu.SemaphoreType.DMA,
            pltpu.SemaphoreType.DMA,
        ],
        compiler_params=pltpu.CompilerParams(needs_layout_passes=False,
                                             use_tc_tiling_on_sc=False),
        interpret=interpret,
    )
    return f(h128, src, dst, attr_flat, ew2, eb2)


def _pool_phase(h0t, h1t, batch, G, *, interpret=False):
    N = h0t.shape[0]
    W = _chunk(N, 1000)
    nwin_total = N // W
    ch = _chunk8(G, min(1000, W))
    nq = G // ch

    mesh = plsc.VectorSubcoreMesh(core_axis_name="c", subcore_axis_name="s",
                                  num_cores=NC, num_subcores=NS)

    def body(h0_hbm, h1_hbm, b_hbm, p0, p1, cnt, idx_b, row_v, one_v,
             psum_sh, cnt_sh):
        c = lax.axis_index("c")
        s = lax.axis_index("s")

        zero16 = jnp.zeros((DH,), jnp.float32)
        one16 = jnp.ones((DH,), jnp.float32)

        def fill(i, carry):
            row_v[i] = zero16
            one_v[i] = one16
            return carry
        lax.fori_loop(0, W, fill, 0)
        nq_s = lax.div(jnp.int32(nq) - s + jnp.int32(NS) - 1, jnp.int32(NS))

        def zchunk(k, carry):
            sl = pl.ds(pl.multiple_of((s + k * NS) * ch, 8), ch)
            pltpu.sync_copy(row_v.at[pl.ds(0, ch)], psum_sh.at[sl])
            @pl.when(c == 0)
            def _():
                pltpu.sync_copy(row_v.at[pl.ds(0, ch)], cnt_sh.at[sl])
            return carry
        lax.fori_loop(0, nq_s, zchunk, 0)
        plsc.subcore_barrier()

        nw = lax.div(jnp.int32(nwin_total) - s + jnp.int32(NS) - 1, jnp.int32(NS))

        def window(k, carry):
            wid = s + k * NS
            base = pl.multiple_of(wid * W, 8)
            pltpu.sync_copy(b_hbm.at[pl.ds(base, W)], idx_b)
            @pl.when(c == 0)
            def _():
                pltpu.sync_copy(h0_hbm.at[pl.ds(base, W)], row_v)
            @pl.when(c == 1)
            def _():
                pltpu.sync_copy(h1_hbm.at[pl.ds(base, W)], row_v)
            pltpu.sync_copy(row_v, psum_sh.at[idx_b], add=True)
            @pl.when(c == 0)
            def _():
                pltpu.sync_copy(one_v, cnt_sh.at[idx_b], add=True)
            return carry
        lax.fori_loop(0, nw, window, 0)
        plsc.subcore_barrier()

        def fchunk(k, carry):
            sl = pl.ds(pl.multiple_of((s + k * NS) * ch, 8), ch)
            pltpu.sync_copy(psum_sh.at[sl], row_v.at[pl.ds(0, ch)])
            @pl.when(c == 0)
            def _():
                pltpu.sync_copy(row_v.at[pl.ds(0, ch)], p0.at[sl])
                pltpu.sync_copy(cnt_sh.at[sl], one_v.at[pl.ds(0, ch)])
                pltpu.sync_copy(one_v.at[pl.ds(0, ch)], cnt.at[sl])
            @pl.when(c == 1)
            def _():
                pltpu.sync_copy(row_v.at[pl.ds(0, ch)], p1.at[sl])
            return carry
        lax.fori_loop(0, nq_s, fchunk, 0)

    f = pl.kernel(
        body,
        out_type=(jax.ShapeDtypeStruct((G, DH), jnp.float32),
                  jax.ShapeDtypeStruct((G, DH), jnp.float32),
                  jax.ShapeDtypeStruct((G, DH), jnp.float32)),
        mesh=mesh,
        scratch_types=[
            pltpu.VMEM((W,), jnp.int32),
            pltpu.VMEM((W, DH), jnp.float32),
            pltpu.VMEM((W, DH), jnp.float32),
            pltpu.VMEM_SHARED((G, DH), jnp.float32),
            pltpu.VMEM_SHARED((G, DH), jnp.float32),
        ],
        compiler_params=pltpu.CompilerParams(use_tc_tiling_on_sc=False),
        interpret=interpret,
    )
    return f(h0t, h1t, batch)


def _embed_tc(x, node_w, node_b, *, interpret=False):
    N, F = x.shape
    D = node_w.shape[1]
    B = _chunk(N, 10000)

    def body(x_ref, w_ref, b_ref, o_ref):
        h = jnp.dot(x_ref[...], w_ref[...],
                    preferred_element_type=jnp.float32) + b_ref[...]
        o_ref[...] = jnp.concatenate(
            [h, jnp.zeros((B, 128 - D), jnp.float32)], axis=1)

    return pl.pallas_call(
        body,
        grid=(N // B,),
        in_specs=[pl.BlockSpec((B, F), lambda i: (i, 0)),
                  pl.BlockSpec((F, D), lambda i: (0, 0)),
                  pl.BlockSpec((1, D), lambda i: (0, 0))],
        out_specs=pl.BlockSpec((B, 128), lambda i: (i, 0)),
        out_shape=jax.ShapeDtypeStruct((N, 128), jnp.float32),
        interpret=interpret,
    )(x, node_w, node_b.reshape(1, D))


def _mlp_stats_tc(h, a0, a1, w1, b1, w2, b2, *, interpret=False):
    N = h.shape[0]
    D = 2 * DH
    H = w1.shape[1]
    B = _chunk(N, 10000)

    def body(h_ref, a0_ref, a1_ref, w1_ref, b1_ref, w2_ref, b2_ref,
             z_ref, s_ref, q_ref):
        i = pl.program_id(0)
        xx = h_ref[:, :D] + jnp.concatenate([a0_ref[...], a1_ref[...]], axis=1)
        t = jnp.maximum(jnp.dot(xx, w1_ref[...],
                                preferred_element_type=jnp.float32)
                        + b1_ref[...], 0.0)
        z = jnp.dot(t, w2_ref[...], preferred_element_type=jnp.float32) \
            + b2_ref[...]
        z_ref[...] = z

        @pl.when(i == 0)
        def _():
            s_ref[...] = jnp.zeros_like(s_ref)
            q_ref[...] = jnp.zeros_like(q_ref)
        s_ref[...] += jnp.sum(z, axis=0, keepdims=True)
        q_ref[...] += jnp.sum(z * z, axis=0, keepdims=True)

    return pl.pallas_call(
        body,
        grid=(N // B,),
        in_specs=[pl.BlockSpec((B, 128), lambda i: (i, 0)),
                  pl.BlockSpec((B, DH), lambda i: (i, 0)),
                  pl.BlockSpec((B, DH), lambda i: (i, 0)),
                  pl.BlockSpec((D, H), lambda i: (0, 0)),
                  pl.BlockSpec((1, H), lambda i: (0, 0)),
                  pl.BlockSpec((H, D), lambda i: (0, 0)),
                  pl.BlockSpec((1, D), lambda i: (0, 0))],
        out_specs=(pl.BlockSpec((B, D), lambda i: (i, 0)),
                   pl.BlockSpec((1, D), lambda i: (0, 0)),
                   pl.BlockSpec((1, D), lambda i: (0, 0))),
        out_shape=(jax.ShapeDtypeStruct((N, D), jnp.float32),
                   jax.ShapeDtypeStruct((1, D), jnp.float32),
                   jax.ShapeDtypeStruct((1, D), jnp.float32)),
        interpret=interpret,
    )(h, a0, a1, w1, b1.reshape(1, H), w2, b2.reshape(1, D))


def _bn_relu_tc(z, zsum, zsq, g, bt, n_rows, *, interpret=False):
    N, D = z.shape
    B = _chunk(N, 10000)

    def body(z_ref, s_ref, q_ref, g_ref, b_ref, o_ref, o0_ref, o1_ref):
        inv_n = jnp.float32(1.0 / n_rows)
        mean = s_ref[...] * inv_n
        var = q_ref[...] * inv_n - mean * mean
        scale = g_ref[...] * lax.rsqrt(var + EPS_BN_)
        shift = b_ref[...] - mean * scale
        h = jnp.maximum(z_ref[...] * scale + shift, 0.0)
        o_ref[...] = jnp.concatenate(
            [h, jnp.zeros((B, 128 - D), jnp.float32)], axis=1)
        o0_ref[...] = h[:, :DH]
        o1_ref[...] = h[:, DH:]

    return pl.pallas_call(
        body,
        grid=(N // B,),
        in_specs=[pl.BlockSpec((B, D), lambda i: (i, 0)),
                  pl.BlockSpec((1, D), lambda i: (0, 0)),
                  pl.BlockSpec((1, D), lambda i: (0, 0)),
                  pl.BlockSpec((1, D), lambda i: (0, 0)),
                  pl.BlockSpec((1, D), lambda i: (0, 0))],
        out_specs=(pl.BlockSpec((B, 128), lambda i: (i, 0)),
                   pl.BlockSpec((B, DH), lambda i: (i, 0)),
                   pl.BlockSpec((B, DH), lambda i: (i, 0))),
        out_shape=(jax.ShapeDtypeStruct((N, 128), jnp.float32),
                   jax.ShapeDtypeStruct((N, DH), jnp.float32),
                   jax.ShapeDtypeStruct((N, DH), jnp.float32)),
        interpret=interpret,
    )(z, zsum, zsq, g.reshape(1, D), bt.reshape(1, D))


def _head_tc(p0, p1, cnt, l1w, l1b, l2w, l2b, *, interpret=False):
    G = p0.shape[0]
    D = 2 * DH
    H = l1w.shape[1]
    O = l2w.shape[1]

    def body(p0_ref, p1_ref, c_ref, w1_ref, b1_ref, w2_ref, b2_ref, o_ref):
        ssum = jnp.concatenate([p0_ref[...], p1_ref[...]], axis=1)
        c = jnp.maximum(c_ref[...][:, 0:1], 1.0)
        gx = ssum / c
        t = jnp.maximum(jnp.dot(gx, w1_ref[...],
                                preferred_element_type=jnp.float32)
                        + b1_ref[...], 0.0)
        o_ref[...] = jnp.dot(t, w2_ref[...],
                             preferred_element_type=jnp.float32) + b2_ref[...]

    return pl.pallas_call(
        body,
        out_shape=jax.ShapeDtypeStruct((G, O), jnp.float32),
        interpret=interpret,
    )(p0, p1, cnt, l1w, l1b.reshape(1, H), l2w, l2b.reshape(1, O))


def _forward(x, edge_index, edge_attr, batch,
             node_w, node_b, edge_w, edge_b,
             conv0_w1, conv0_b1, conv0_w2, conv0_b2, bn0_g, bn0_b,
             conv1_w1, conv1_b1, conv1_w2, conv1_b2, bn1_g, bn1_b,
             lin1_w, lin1_b, lin2_w, lin2_b, G, interpret=False):
    N = x.shape[0]
    src = edge_index[0]
    dst = edge_index[1]
    ew2 = edge_w.reshape(3, 2, DH).transpose(1, 0, 2).reshape(6 * DH)
    eb2 = edge_b.reshape(2 * DH)
    attr_flat = edge_attr.reshape(-1)

    h128 = _embed_tc(x, node_w, node_b, interpret=interpret)
    layers = [
        (conv0_w1, conv0_b1, conv0_w2, conv0_b2, bn0_g, bn0_b),
        (conv1_w1, conv1_b1, conv1_w2, conv1_b2, bn1_g, bn1_b),
    ]
    h0t = h1t = None
    for (w1, b1, w2, b2, g, bt) in layers:
        a0, a1 = _edge_phase(h128, src, dst, attr_flat, ew2, eb2,
                             interpret=interpret)
        z, zsum, zsq = _mlp_stats_tc(h128, a0, a1, w1, b1, w2, b2,
                                     interpret=interpret)
        h128, h0t, h1t = _bn_relu_tc(z, zsum, zsq, g, bt, N,
                                     interpret=interpret)

    p0, p1, cnt = _pool_phase(h0t, h1t, batch, G, interpret=interpret)
    return _head_tc(p0, p1, cnt, lin1_w, lin1_b, lin2_w, lin2_b,
                    interpret=interpret)


def kernel(x, edge_index, edge_attr, batch,
           node_w, node_b, edge_w, edge_b,
           conv0_w1, conv0_b1, conv0_w2, conv0_b2, bn0_g, bn0_b,
           conv1_w1, conv1_b1, conv1_w2, conv1_b2, bn1_g, bn1_b,
           lin1_w, lin1_b, lin2_w, lin2_b):
    return _forward(x, edge_index, edge_attr, batch,
                    node_w, node_b, edge_w, edge_b,
                    conv0_w1, conv0_b1, conv0_w2, conv0_b2, bn0_g, bn0_b,
                    conv1_w1, conv1_b1, conv1_w2, conv1_b2, bn1_g, bn1_b,
                    lin1_w, lin1_b, lin2_w, lin2_b, G=2000)


# speedup vs baseline: 1.0103x; 1.0103x over previous
import functools

import jax
import jax.numpy as jnp
from jax import lax
from jax.experimental import pallas as pl
from jax.experimental.pallas import tpu as pltpu
from jax.experimental.pallas import tpu_sc as plsc

NC = 2
NS = 16
DH = 16
EPS_BN_ = 1e-5


def _chunk(rows, cap):
    ch = min(rows, cap)
    while rows % ch:
        ch -= 1
    return ch


def _chunk8(total, cap):
    cand = [d for d in range(8, cap + 1, 8) if total % d == 0]
    assert cand, (total, cap)
    good = [d for d in cand if total // d >= NS]
    return max(good) if good else min(cand)


def _split_sc(h_flat, N, *, interpret=False):
    CH = 250
    nq = N // CH
    mesh = plsc.VectorSubcoreMesh(core_axis_name="c", subcore_axis_name="s",
                                  num_cores=NC, num_subcores=NS)

    def body(hf_hbm, out0, out1, x_v, o0_v, o1_v):
        c = lax.axis_index("c")
        s = lax.axis_index("s")
        wid = c * NS + s
        nw = lax.div(jnp.int32(nq) - wid + jnp.int32(NC * NS) - 1,
                     jnp.int32(NC * NS))

        def chunk(k, carry):
            q = wid + k * (NC * NS)
            off = pl.multiple_of(q * (CH * 128), 8)
            pltpu.sync_copy(hf_hbm.at[pl.ds(off, CH * 128)], x_v)

            def row(r, icarry):
                o0_v[r] = x_v[pl.ds(r * 128, DH)]
                o1_v[r] = x_v[pl.ds(r * 128 + DH, DH)]
                return icarry
            lax.fori_loop(0, CH, row, 0, unroll=4)
            sl = pl.ds(pl.multiple_of(q * CH, 2), CH)
            pltpu.sync_copy(o0_v, out0.at[sl])
            pltpu.sync_copy(o1_v, out1.at[sl])
            return carry
        lax.fori_loop(0, nw, chunk, 0)

    f = pl.kernel(
        body,
        out_type=(jax.ShapeDtypeStruct((N, DH), jnp.float32),
                  jax.ShapeDtypeStruct((N, DH), jnp.float32)),
        mesh=mesh,
        scratch_types=[
            pltpu.VMEM((CH * 128,), jnp.float32),
            pltpu.VMEM((CH, DH), jnp.float32),
            pltpu.VMEM((CH, DH), jnp.float32),
        ],
        compiler_params=pltpu.CompilerParams(use_tc_tiling_on_sc=False),
        interpret=interpret,
    )
    return f(h_flat)


def _edge_phase(h0t, h1t, src, dst, attr_flat, ew2, eb2, *, interpret=False):
    N = h0t.shape[0]
    E = src.shape[0]
    ew = E // NS
    W = _chunk(ew, 160)
    assert W % DH == 0 and W % 8 == 0
    nwin = ew // W
    assert nwin % 2 == 0
    ch = _chunk8(N, W)
    nq = N // ch

    mesh = plsc.VectorSubcoreMesh(core_axis_name="c", subcore_axis_name="s",
                                  num_cores=NC, num_subcores=NS)

    def body(h0_hbm, h1_hbm, src_hbm, dst_hbm, attr_hbm, ew_hbm, eb_hbm,
             out0, out1, ew_v, eb_v,
             s_v0, s_v1, d_v0, d_v1, a_v0, a_v1, r_v0, r_v1, m_v0, m_v1,
             sd_v0, sd_v1, aggr_sh, lsem0, lsem1, gsem0, gsem1, csem0, csem1):
        c = lax.axis_index("c")
        s = lax.axis_index("s")
        s_v = [s_v0, s_v1]
        d_v = [d_v0, d_v1]
        a_v = [a_v0, a_v1]
        r_v = [r_v0, r_v1]
        m_v = [m_v0, m_v1]
        sd_v = [sd_v0, sd_v1]
        lsem = [lsem0, lsem1]
        gsem = [gsem0, gsem1]
        csem = [csem0, csem1]

        zero16 = jnp.zeros((DH,), jnp.float32)

        def zrow(i, carry):
            m_v0[i] = zero16
            return carry
        lax.fori_loop(0, ch, zrow, 0)
        nq_s = lax.div(jnp.int32(nq) - s + jnp.int32(NS) - 1, jnp.int32(NS))

        def zchunk(k, carry):
            off = pl.multiple_of((s + k * NS) * ch, 8)
            pltpu.sync_copy(m_v0.at[pl.ds(0, ch)],
                            aggr_sh.at[pl.ds(off, ch)])
            return carry
        lax.fori_loop(0, nq_s, zchunk, 0)
        plsc.subcore_barrier()

        for cc in range(NC):
            @pl.when(c == cc)
            def _():
                pltpu.sync_copy(ew_hbm.at[pl.ds(cc * 3 * DH, 3 * DH)], ew_v)
                pltpu.sync_copy(eb_hbm.at[pl.ds(cc * DH, DH)], eb_v)
        w0 = ew_v[pl.ds(0, DH)]
        w1 = ew_v[pl.ds(DH, DH)]
        w2 = ew_v[pl.ds(2 * DH, DH)]
        bb = eb_v[...]

        base0 = s * ew

        def lin_descs(g, k):
            base = pl.multiple_of(base0 + g * W, 8)
            base3 = pl.multiple_of(base * 3, 8)
            return (
                pltpu.make_async_copy(src_hbm.at[pl.ds(base, W)], s_v[k],
                                      lsem[k]),
                pltpu.make_async_copy(dst_hbm.at[pl.ds(base, W)], d_v[k],
                                      lsem[k]),
                pltpu.make_async_copy(attr_hbm.at[pl.ds(base3, 3 * W)],
                                      a_v[k].at[pl.ds(0, 3 * W)], lsem[k]),
            )

        def issue_lin(g, k):
            for d in lin_descs(g, k):
                d.start()

        def wait_lin(g, k):
            for d in lin_descs(g, k):
                d.wait()

        def gat_start(k):
            @pl.when(c == 0)
            def _():
                pltpu.make_async_copy(h0_hbm.at[s_v[k]], r_v[k],
                                      gsem[k]).start()
            @pl.when(c == 1)
            def _():
                pltpu.make_async_copy(h1_hbm.at[s_v[k]], r_v[k],
                                      gsem[k]).start()

        def gat_wait(k):
            pltpu.make_async_copy(h0_hbm.at[s_v[k]], r_v[k], gsem[k]).wait()

        def sc_desc(k):
            return pltpu.make_async_copy(m_v[k], aggr_sh.at[sd_v[k]], csem[k])

        iota16 = lax.iota(jnp.int32, DH)

        def compute(k):
            for q in range(W // DH):
                jq = iota16 + (q * DH)
                a0 = plsc.load_gather(a_v[k], [jq * 3])
                a1 = plsc.load_gather(a_v[k], [jq * 3 + 1])
                a2 = plsc.load_gather(a_v[k], [jq * 3 + 2])
                for fi in range(DH):
                    lane = jnp.broadcast_to(jnp.int32(fi), (DH,))
                    hf = plsc.load_gather(r_v[k], [jq, lane])
                    ef = a0 * w0[fi] + a1 * w1[fi] + a2 * w2[fi] + bb[fi]
                    m16 = jnp.maximum(hf + ef, 0.0)
                    fv = jnp.broadcast_to(jnp.int32(fi), (DH,))
                    plsc.store_scatter(m_v[k], [jq, fv], m16)
            for q in range(W // DH):
                sd_v[k][pl.ds(q * DH, DH)] = d_v[k][pl.ds(q * DH, DH)]

        def half(t, g, k):
            @pl.when(g + 1 < nwin)
            def _():
                wait_lin(g + 1, k ^ 1)
                gat_start(k ^ 1)
            gat_wait(k)
            compute(k)
            pltpu.sync_copy(m_v[k], aggr_sh.at[sd_v[k]], add=True)
            @pl.when(g + 2 < nwin)
            def _():
                issue_lin(g + 2, k)

        USE_ASYNC = True
        if USE_ASYNC:
            issue_lin(0, 0)
            issue_lin(1, 1)
            wait_lin(0, 0)
            gat_start(0)

            def body2(t, carry):
                half(t, 2 * t, 0)
                half(t, 2 * t + 1, 1)
                return carry
            lax.fori_loop(0, nwin // 2, body2, 0)
        else:
            def win_sync(g, carry):
                issue_lin(g, 0)
                wait_lin(g, 0)
                gat_start(0)
                gat_wait(0)
                compute(0)
                pltpu.sync_copy(m_v[0], aggr_sh.at[sd_v[0]], add=True)
                return carry
            lax.fori_loop(0, nwin, win_sync, 0)
        plsc.subcore_barrier()

        def fchunk(k, carry):
            off = pl.multiple_of((s + k * NS) * ch, 8)
            sl = pl.ds(off, ch)
            pltpu.sync_copy(aggr_sh.at[sl], m_v0.at[pl.ds(0, ch)])
            @pl.when(c == 0)
            def _():
                pltpu.sync_copy(m_v0.at[pl.ds(0, ch)], out0.at[sl])
            @pl.when(c == 1)
            def _():
                pltpu.sync_copy(m_v0.at[pl.ds(0, ch)], out1.at[sl])
            return carry
        lax.fori_loop(0, nq_s, fchunk, 0)

    f = pl.kernel(
        body,
        out_type=(jax.ShapeDtypeStruct((N, DH), jnp.float32),
                  jax.ShapeDtypeStruct((N, DH), jnp.float32)),
        mesh=mesh,
        scratch_types=[
            pltpu.VMEM((3 * DH,), jnp.float32),
            pltpu.VMEM((DH,), jnp.float32),
            pltpu.VMEM((W,), jnp.int32),
            pltpu.VMEM((W,), jnp.int32),
            pltpu.VMEM((W,), jnp.int32),
            pltpu.VMEM((W,), jnp.int32),
            pltpu.VMEM((3 * W + DH,), jnp.float32),
            pltpu.VMEM((3 * W + DH,), jnp.float32),
            pltpu.VMEM((W, DH), jnp.float32),
            pltpu.VMEM((W, DH), jnp.float32),
            pltpu.VMEM((W, DH), jnp.float32),
            pltpu.VMEM((W, DH), jnp.float32),
            pltpu.VMEM((W,), jnp.int32),
            pltpu.VMEM((W,), jnp.int32),
            pltpu.VMEM_SHARED((N, DH), jnp.float32),
            pltpu.SemaphoreType.DMA,
            pltpu.SemaphoreType.DMA,
            pltpu.SemaphoreType.DMA,
            pltpu.SemaphoreType.DMA,
            pltpu.SemaphoreType.DMA,
            pltpu.SemaphoreType.DMA,
        ],
        compiler_params=pltpu.CompilerParams(needs_layout_passes=False,
                                             use_tc_tiling_on_sc=False),
        interpret=interpret,
    )
    return f(h0t, h1t, src, dst, attr_flat, ew2, eb2)


def _pool_phase(h0t, h1t, batch, G, *, interpret=False):
    N = h0t.shape[0]
    W = _chunk(N, 1000)
    nwin_total = N // W
    ch = _chunk8(G, min(1000, W))
    nq = G // ch

    mesh = plsc.VectorSubcoreMesh(core_axis_name="c", subcore_axis_name="s",
                                  num_cores=NC, num_subcores=NS)

    def body(h0_hbm, h1_hbm, b_hbm, p0, p1, cnt, idx_b, row_v, one_v,
             psum_sh, cnt_sh):
        c = lax.axis_index("c")
        s = lax.axis_index("s")

        zero16 = jnp.zeros((DH,), jnp.float32)
        one16 = jnp.ones((DH,), jnp.float32)

        def fill(i, carry):
            row_v[i] = zero16
            one_v[i] = one16
            return carry
        lax.fori_loop(0, W, fill, 0)
        nq_s = lax.div(jnp.int32(nq) - s + jnp.int32(NS) - 1, jnp.int32(NS))

        def zchunk(k, carry):
            sl = pl.ds(pl.multiple_of((s + k * NS) * ch, 8), ch)
            pltpu.sync_copy(row_v.at[pl.ds(0, ch)], psum_sh.at[sl])
            @pl.when(c == 0)
            def _():
                pltpu.sync_copy(row_v.at[pl.ds(0, ch)], cnt_sh.at[sl])
            return carry
        lax.fori_loop(0, nq_s, zchunk, 0)
        plsc.subcore_barrier()

        nw = lax.div(jnp.int32(nwin_total) - s + jnp.int32(NS) - 1, jnp.int32(NS))

        def window(k, carry):
            wid = s + k * NS
            base = pl.multiple_of(wid * W, 8)
            pltpu.sync_copy(b_hbm.at[pl.ds(base, W)], idx_b)
            @pl.when(c == 0)
            def _():
                pltpu.sync_copy(h0_hbm.at[pl.ds(base, W)], row_v)
            @pl.when(c == 1)
            def _():
                pltpu.sync_copy(h1_hbm.at[pl.ds(base, W)], row_v)
            pltpu.sync_copy(row_v, psum_sh.at[idx_b], add=True)
            @pl.when(c == 0)
            def _():
                pltpu.sync_copy(one_v, cnt_sh.at[idx_b], add=True)
            return carry
        lax.fori_loop(0, nw, window, 0)
        plsc.subcore_barrier()

        def fchunk(k, carry):
            sl = pl.ds(pl.multiple_of((s + k * NS) * ch, 8), ch)
            pltpu.sync_copy(psum_sh.at[sl], row_v.at[pl.ds(0, ch)])
            @pl.when(c == 0)
            def _():
                pltpu.sync_copy(row_v.at[pl.ds(0, ch)], p0.at[sl])
                pltpu.sync_copy(cnt_sh.at[sl], one_v.at[pl.ds(0, ch)])
                pltpu.sync_copy(one_v.at[pl.ds(0, ch)], cnt.at[sl])
            @pl.when(c == 1)
            def _():
                pltpu.sync_copy(row_v.at[pl.ds(0, ch)], p1.at[sl])
            return carry
        lax.fori_loop(0, nq_s, fchunk, 0)

    f = pl.kernel(
        body,
        out_type=(jax.ShapeDtypeStruct((G, DH), jnp.float32),
                  jax.ShapeDtypeStruct((G, DH), jnp.float32),
                  jax.ShapeDtypeStruct((G, DH), jnp.float32)),
        mesh=mesh,
        scratch_types=[
            pltpu.VMEM((W,), jnp.int32),
            pltpu.VMEM((W, DH), jnp.float32),
            pltpu.VMEM((W, DH), jnp.float32),
            pltpu.VMEM_SHARED((G, DH), jnp.float32),
            pltpu.VMEM_SHARED((G, DH), jnp.float32),
        ],
        compiler_params=pltpu.CompilerParams(use_tc_tiling_on_sc=False),
        interpret=interpret,
    )
    return f(h0t, h1t, batch)


def _embed_tc(x, node_w, node_b, *, interpret=False):
    N, F = x.shape
    D = node_w.shape[1]
    B = _chunk(N, 10000)

    def body(x_ref, w_ref, b_ref, o_ref):
        h = jnp.dot(x_ref[...], w_ref[...],
                    preferred_element_type=jnp.float32) + b_ref[...]
        o_ref[...] = jnp.concatenate(
            [h, jnp.zeros((B, 128 - D), jnp.float32)], axis=1)

    return pl.pallas_call(
        body,
        grid=(N // B,),
        in_specs=[pl.BlockSpec((B, F), lambda i: (i, 0)),
                  pl.BlockSpec((F, D), lambda i: (0, 0)),
                  pl.BlockSpec((1, D), lambda i: (0, 0))],
        out_specs=pl.BlockSpec((B, 128), lambda i: (i, 0)),
        out_shape=jax.ShapeDtypeStruct((N, 128), jnp.float32),
        interpret=interpret,
    )(x, node_w, node_b.reshape(1, D))


def _mlp_stats_tc(h, a0, a1, w1, b1, w2, b2, *, interpret=False):
    N = h.shape[0]
    D = 2 * DH
    H = w1.shape[1]
    B = _chunk(N, 10000)

    def body(h_ref, a0_ref, a1_ref, w1_ref, b1_ref, w2_ref, b2_ref,
             z_ref, s_ref, q_ref):
        i = pl.program_id(0)
        xx = h_ref[:, :D] + jnp.concatenate([a0_ref[...], a1_ref[...]], axis=1)
        t = jnp.maximum(jnp.dot(xx, w1_ref[...],
                                preferred_element_type=jnp.float32)
                        + b1_ref[...], 0.0)
        z = jnp.dot(t, w2_ref[...], preferred_element_type=jnp.float32) \
            + b2_ref[...]
        z_ref[...] = z

        @pl.when(i == 0)
        def _():
            s_ref[...] = jnp.zeros_like(s_ref)
            q_ref[...] = jnp.zeros_like(q_ref)
        s_ref[...] += jnp.sum(z, axis=0, keepdims=True)
        q_ref[...] += jnp.sum(z * z, axis=0, keepdims=True)

    return pl.pallas_call(
        body,
        grid=(N // B,),
        in_specs=[pl.BlockSpec((B, 128), lambda i: (i, 0)),
                  pl.BlockSpec((B, DH), lambda i: (i, 0)),
                  pl.BlockSpec((B, DH), lambda i: (i, 0)),
                  pl.BlockSpec((D, H), lambda i: (0, 0)),
                  pl.BlockSpec((1, H), lambda i: (0, 0)),
                  pl.BlockSpec((H, D), lambda i: (0, 0)),
                  pl.BlockSpec((1, D), lambda i: (0, 0))],
        out_specs=(pl.BlockSpec((B, D), lambda i: (i, 0)),
                   pl.BlockSpec((1, D), lambda i: (0, 0)),
                   pl.BlockSpec((1, D), lambda i: (0, 0))),
        out_shape=(jax.ShapeDtypeStruct((N, D), jnp.float32),
                   jax.ShapeDtypeStruct((1, D), jnp.float32),
                   jax.ShapeDtypeStruct((1, D), jnp.float32)),
        interpret=interpret,
    )(h, a0, a1, w1, b1.reshape(1, H), w2, b2.reshape(1, D))


def _bn_relu_tc(z, zsum, zsq, g, bt, n_rows, *, interpret=False):
    N, D = z.shape
    B = _chunk(N, 10000)

    def body(z_ref, s_ref, q_ref, g_ref, b_ref, o_ref):
        inv_n = jnp.float32(1.0 / n_rows)
        mean = s_ref[...] * inv_n
        var = q_ref[...] * inv_n - mean * mean
        scale = g_ref[...] * lax.rsqrt(var + EPS_BN_)
        shift = b_ref[...] - mean * scale
        h = jnp.maximum(z_ref[...] * scale + shift, 0.0)
        o_ref[...] = jnp.concatenate(
            [h, jnp.zeros((B, 128 - D), jnp.float32)], axis=1)

    return pl.pallas_call(
        body,
        grid=(N // B,),
        in_specs=[pl.BlockSpec((B, D), lambda i: (i, 0)),
                  pl.BlockSpec((1, D), lambda i: (0, 0)),
                  pl.BlockSpec((1, D), lambda i: (0, 0)),
                  pl.BlockSpec((1, D), lambda i: (0, 0)),
                  pl.BlockSpec((1, D), lambda i: (0, 0))],
        out_specs=pl.BlockSpec((B, 128), lambda i: (i, 0)),
        out_shape=jax.ShapeDtypeStruct((N, 128), jnp.float32),
        interpret=interpret,
    )(z, zsum, zsq, g.reshape(1, D), bt.reshape(1, D))


def _head_tc(p0, p1, cnt, l1w, l1b, l2w, l2b, *, interpret=False):
    G = p0.shape[0]
    D = 2 * DH
    H = l1w.shape[1]
    O = l2w.shape[1]

    def body(p0_ref, p1_ref, c_ref, w1_ref, b1_ref, w2_ref, b2_ref, o_ref):
        ssum = jnp.concatenate([p0_ref[...], p1_ref[...]], axis=1)
        c = jnp.maximum(c_ref[...][:, 0:1], 1.0)
        gx = ssum / c
        t = jnp.maximum(jnp.dot(gx, w1_ref[...],
                                preferred_element_type=jnp.float32)
                        + b1_ref[...], 0.0)
        o_ref[...] = jnp.dot(t, w2_ref[...],
                             preferred_element_type=jnp.float32) + b2_ref[...]

    return pl.pallas_call(
        body,
        out_shape=jax.ShapeDtypeStruct((G, O), jnp.float32),
        interpret=interpret,
    )(p0, p1, cnt, l1w, l1b.reshape(1, H), l2w, l2b.reshape(1, O))


def _forward(x, edge_index, edge_attr, batch,
             node_w, node_b, edge_w, edge_b,
             conv0_w1, conv0_b1, conv0_w2, conv0_b2, bn0_g, bn0_b,
             conv1_w1, conv1_b1, conv1_w2, conv1_b2, bn1_g, bn1_b,
             lin1_w, lin1_b, lin2_w, lin2_b, G, interpret=False):
    N = x.shape[0]
    src = edge_index[0]
    dst = edge_index[1]
    ew2 = edge_w.reshape(3, 2, DH).transpose(1, 0, 2).reshape(6 * DH)
    eb2 = edge_b.reshape(2 * DH)
    attr_flat = edge_attr.reshape(-1)

    h128 = _embed_tc(x, node_w, node_b, interpret=interpret)
    layers = [
        (conv0_w1, conv0_b1, conv0_w2, conv0_b2, bn0_g, bn0_b),
        (conv1_w1, conv1_b1, conv1_w2, conv1_b2, bn1_g, bn1_b),
    ]
    for (w1, b1, w2, b2, g, bt) in layers:
        h0t, h1t = _split_sc(h128.reshape(-1), N, interpret=interpret)
        a0, a1 = _edge_phase(h0t, h1t, src, dst, attr_flat, ew2, eb2,
                             interpret=interpret)
        z, zsum, zsq = _mlp_stats_tc(h128, a0, a1, w1, b1, w2, b2,
                                     interpret=interpret)
        h128 = _bn_relu_tc(z, zsum, zsq, g, bt, N, interpret=interpret)

    h0t, h1t = _split_sc(h128.reshape(-1), N, interpret=interpret)
    p0, p1, cnt = _pool_phase(h0t, h1t, batch, G, interpret=interpret)
    return _head_tc(p0, p1, cnt, lin1_w, lin1_b, lin2_w, lin2_b,
                    interpret=interpret)


def kernel(x, edge_index, edge_attr, batch,
           node_w, node_b, edge_w, edge_b,
           conv0_w1, conv0_b1, conv0_w2, conv0_b2, bn0_g, bn0_b,
           conv1_w1, conv1_b1, conv1_w2, conv1_b2, bn1_g, bn1_b,
           lin1_w, lin1_b, lin2_w, lin2_b):
    return _forward(x, edge_index, edge_attr, batch,
                    node_w, node_b, edge_w, edge_b,
                    conv0_w1, conv0_b1, conv0_w2, conv0_b2, bn0_g, bn0_b,
                    conv1_w1, conv1_b1, conv1_w2, conv1_b2, bn1_g, bn1_b,
                    lin1_w, lin1_b, lin2_w, lin2_b, G=2000)


# speedup vs baseline: 1.0360x; 1.0255x over previous
import functools

import jax
import jax.numpy as jnp
from jax import lax
from jax.experimental import pallas as pl
from jax.experimental.pallas import tpu as pltpu
from jax.experimental.pallas import tpu_sc as plsc

NC = 2
NS = 16
DH = 16
EPS_BN_ = 1e-5


def _chunk(rows, cap):
    ch = min(rows, cap)
    while rows % ch:
        ch -= 1
    return ch


def _chunk8(total, cap):
    cand = [d for d in range(8, cap + 1, 8) if total % d == 0]
    assert cand, (total, cap)
    good = [d for d in cand if total // d >= NS]
    return max(good) if good else min(cand)


def _split_sc(h_flat, N, *, interpret=False):
    CH = 250
    nq = N // CH
    mesh = plsc.VectorSubcoreMesh(core_axis_name="c", subcore_axis_name="s",
                                  num_cores=NC, num_subcores=NS)

    def body(hf_hbm, out0, out1, x_v, o0_v, o1_v):
        c = lax.axis_index("c")
        s = lax.axis_index("s")
        wid = c * NS + s
        nw = lax.div(jnp.int32(nq) - wid + jnp.int32(NC * NS) - 1,
                     jnp.int32(NC * NS))

        def chunk(k, carry):
            q = wid + k * (NC * NS)
            off = pl.multiple_of(q * (CH * 128), 8)
            pltpu.sync_copy(hf_hbm.at[pl.ds(off, CH * 128)], x_v)

            def row(r, icarry):
                o0_v[r] = x_v[pl.ds(r * 128, DH)]
                o1_v[r] = x_v[pl.ds(r * 128 + DH, DH)]
                return icarry
            lax.fori_loop(0, CH, row, 0, unroll=4)
            sl = pl.ds(pl.multiple_of(q * CH, 2), CH)
            pltpu.sync_copy(o0_v, out0.at[sl])
            pltpu.sync_copy(o1_v, out1.at[sl])
            return carry
        lax.fori_loop(0, nw, chunk, 0)

    f = pl.kernel(
        body,
        out_type=(jax.ShapeDtypeStruct((N, DH), jnp.float32),
                  jax.ShapeDtypeStruct((N, DH), jnp.float32)),
        mesh=mesh,
        scratch_types=[
            pltpu.VMEM((CH * 128,), jnp.float32),
            pltpu.VMEM((CH, DH), jnp.float32),
            pltpu.VMEM((CH, DH), jnp.float32),
        ],
        compiler_params=pltpu.CompilerParams(use_tc_tiling_on_sc=False),
        interpret=interpret,
    )
    return f(h_flat)


def _edge_phase(h0t, h1t, src, dst, attr_flat, ew2, eb2, *, interpret=False):
    N = h0t.shape[0]
    E = src.shape[0]
    ew = E // NS
    W = _chunk(ew, 160)
    assert W % DH == 0 and W % 8 == 0
    nwin = ew // W
    assert nwin % 2 == 0
    ch = _chunk8(N, W)
    nq = N // ch

    mesh = plsc.VectorSubcoreMesh(core_axis_name="c", subcore_axis_name="s",
                                  num_cores=NC, num_subcores=NS)

    def body(h0_hbm, h1_hbm, src_hbm, dst_hbm, attr_hbm, ew_hbm, eb_hbm,
             out0, out1, ew_v, eb_v,
             s_v0, s_v1, d_v0, d_v1, a_v0, a_v1, r_v0, r_v1, m_v, sd_v,
             aggr_sh, lsem0, lsem1, gsem0, gsem1):
        c = lax.axis_index("c")
        s = lax.axis_index("s")
        s_v = [s_v0, s_v1]
        d_v = [d_v0, d_v1]
        a_v = [a_v0, a_v1]
        r_v = [r_v0, r_v1]
        lsem = [lsem0, lsem1]
        gsem = [gsem0, gsem1]

        zero16 = jnp.zeros((DH,), jnp.float32)

        def zrow(i, carry):
            m_v[i] = zero16
            return carry
        lax.fori_loop(0, ch, zrow, 0)
        nq_s = lax.div(jnp.int32(nq) - s + jnp.int32(NS) - 1, jnp.int32(NS))

        def zchunk(k, carry):
            off = pl.multiple_of((s + k * NS) * ch, 8)
            pltpu.sync_copy(m_v.at[pl.ds(0, ch)],
                            aggr_sh.at[pl.ds(off, ch)])
            return carry
        lax.fori_loop(0, nq_s, zchunk, 0)
        plsc.subcore_barrier()

        for cc in range(NC):
            @pl.when(c == cc)
            def _():
                pltpu.sync_copy(ew_hbm.at[pl.ds(cc * 3 * DH, 3 * DH)], ew_v)
                pltpu.sync_copy(eb_hbm.at[pl.ds(cc * DH, DH)], eb_v)
        w0 = ew_v[pl.ds(0, DH)]
        w1 = ew_v[pl.ds(DH, DH)]
        w2 = ew_v[pl.ds(2 * DH, DH)]
        bb = eb_v[...]

        base0 = s * ew

        def lin_descs(g, k):
            base = pl.multiple_of(base0 + g * W, 8)
            base3 = pl.multiple_of(base * 3, 8)
            return (
                pltpu.make_async_copy(src_hbm.at[pl.ds(base, W)], s_v[k],
                                      lsem[k]),
                pltpu.make_async_copy(dst_hbm.at[pl.ds(base, W)], d_v[k],
                                      lsem[k]),
                pltpu.make_async_copy(attr_hbm.at[pl.ds(base3, 3 * W)],
                                      a_v[k].at[pl.ds(0, 3 * W)], lsem[k]),
            )

        def issue_lin(g, k):
            for d in lin_descs(g, k):
                d.start()

        def wait_lin(g, k):
            for d in lin_descs(g, k):
                d.wait()

        def gat_start(k):
            @pl.when(c == 0)
            def _():
                pltpu.make_async_copy(h0_hbm.at[s_v[k]], r_v[k],
                                      gsem[k]).start()
            @pl.when(c == 1)
            def _():
                pltpu.make_async_copy(h1_hbm.at[s_v[k]], r_v[k],
                                      gsem[k]).start()

        def gat_wait(k):
            pltpu.make_async_copy(h0_hbm.at[s_v[k]], r_v[k], gsem[k]).wait()

        iota16 = lax.iota(jnp.int32, DH)

        def compute(k):
            off = k * W
            for q in range(W // DH):
                jq = iota16 + (q * DH)
                a0 = plsc.load_gather(a_v[k], [jq * 3])
                a1 = plsc.load_gather(a_v[k], [jq * 3 + 1])
                a2 = plsc.load_gather(a_v[k], [jq * 3 + 2])
                jo = jq + off
                for fi in range(DH):
                    lane = jnp.broadcast_to(jnp.int32(fi), (DH,))
                    hf = plsc.load_gather(r_v[k], [jq, lane])
                    ef = a0 * w0[fi] + a1 * w1[fi] + a2 * w2[fi] + bb[fi]
                    m16 = jnp.maximum(hf + ef, 0.0)
                    plsc.store_scatter(m_v, [jo, lane], m16)
            for q in range(W // DH):
                sd_v[pl.ds(off + q * DH, DH)] = d_v[k][pl.ds(q * DH, DH)]

        def half(g, k):
            @pl.when(g + 1 < nwin)
            def _():
                wait_lin(g + 1, k ^ 1)
                gat_start(k ^ 1)
            gat_wait(k)
            compute(k)
            @pl.when(g + 2 < nwin)
            def _():
                issue_lin(g + 2, k)

        issue_lin(0, 0)
        issue_lin(1, 1)
        wait_lin(0, 0)
        gat_start(0)

        def body2(t, carry):
            half(2 * t, 0)
            half(2 * t + 1, 1)
            pltpu.sync_copy(m_v, aggr_sh.at[sd_v], add=True)
            return carry
        lax.fori_loop(0, nwin // 2, body2, 0)
        plsc.subcore_barrier()

        def fchunk(k, carry):
            off = pl.multiple_of((s + k * NS) * ch, 8)
            sl = pl.ds(off, ch)
            pltpu.sync_copy(aggr_sh.at[sl], m_v.at[pl.ds(0, ch)])
            @pl.when(c == 0)
            def _():
                pltpu.sync_copy(m_v.at[pl.ds(0, ch)], out0.at[sl])
            @pl.when(c == 1)
            def _():
                pltpu.sync_copy(m_v.at[pl.ds(0, ch)], out1.at[sl])
            return carry
        lax.fori_loop(0, nq_s, fchunk, 0)

    f = pl.kernel(
        body,
        out_type=(jax.ShapeDtypeStruct((N, DH), jnp.float32),
                  jax.ShapeDtypeStruct((N, DH), jnp.float32)),
        mesh=mesh,
        scratch_types=[
            pltpu.VMEM((3 * DH,), jnp.float32),
            pltpu.VMEM((DH,), jnp.float32),
            pltpu.VMEM((W,), jnp.int32),
            pltpu.VMEM((W,), jnp.int32),
            pltpu.VMEM((W,), jnp.int32),
            pltpu.VMEM((W,), jnp.int32),
            pltpu.VMEM((3 * W + DH,), jnp.float32),
            pltpu.VMEM((3 * W + DH,), jnp.float32),
            pltpu.VMEM((W, DH), jnp.float32),
            pltpu.VMEM((W, DH), jnp.float32),
            pltpu.VMEM((2 * W, DH), jnp.float32),
            pltpu.VMEM((2 * W,), jnp.int32),
            pltpu.VMEM_SHARED((N, DH), jnp.float32),
            pltpu.SemaphoreType.DMA,
            pltpu.SemaphoreType.DMA,
            pltpu.SemaphoreType.DMA,
            pltpu.SemaphoreType.DMA,
        ],
        compiler_params=pltpu.CompilerParams(needs_layout_passes=False,
                                             use_tc_tiling_on_sc=False),
        interpret=interpret,
    )
    return f(h0t, h1t, src, dst, attr_flat, ew2, eb2)


def _pool_phase(h0t, h1t, batch, G, *, interpret=False):
    N = h0t.shape[0]
    W = _chunk(N, 1000)
    nwin_total = N // W
    ch = _chunk8(G, min(1000, W))
    nq = G // ch

    mesh = plsc.VectorSubcoreMesh(core_axis_name="c", subcore_axis_name="s",
                                  num_cores=NC, num_subcores=NS)

    def body(h0_hbm, h1_hbm, b_hbm, p0, p1, cnt, idx_b, row_v, one_v,
             psum_sh, cnt_sh):
        c = lax.axis_index("c")
        s = lax.axis_index("s")

        zero16 = jnp.zeros((DH,), jnp.float32)
        one16 = jnp.ones((DH,), jnp.float32)

        def fill(i, carry):
            row_v[i] = zero16
            one_v[i] = one16
            return carry
        lax.fori_loop(0, W, fill, 0)
        nq_s = lax.div(jnp.int32(nq) - s + jnp.int32(NS) - 1, jnp.int32(NS))

        def zchunk(k, carry):
            sl = pl.ds(pl.multiple_of((s + k * NS) * ch, 8), ch)
            pltpu.sync_copy(row_v.at[pl.ds(0, ch)], psum_sh.at[sl])
            @pl.when(c == 0)
            def _():
                pltpu.sync_copy(row_v.at[pl.ds(0, ch)], cnt_sh.at[sl])
            return carry
        lax.fori_loop(0, nq_s, zchunk, 0)
        plsc.subcore_barrier()

        nw = lax.div(jnp.int32(nwin_total) - s + jnp.int32(NS) - 1, jnp.int32(NS))

        def window(k, carry):
            wid = s + k * NS
            base = pl.multiple_of(wid * W, 8)
            pltpu.sync_copy(b_hbm.at[pl.ds(base, W)], idx_b)
            @pl.when(c == 0)
            def _():
                pltpu.sync_copy(h0_hbm.at[pl.ds(base, W)], row_v)
            @pl.when(c == 1)
            def _():
                pltpu.sync_copy(h1_hbm.at[pl.ds(base, W)], row_v)
            pltpu.sync_copy(row_v, psum_sh.at[idx_b], add=True)
            @pl.when(c == 0)
            def _():
                pltpu.sync_copy(one_v, cnt_sh.at[idx_b], add=True)
            return carry
        lax.fori_loop(0, nw, window, 0)
        plsc.subcore_barrier()

        def fchunk(k, carry):
            sl = pl.ds(pl.multiple_of((s + k * NS) * ch, 8), ch)
            pltpu.sync_copy(psum_sh.at[sl], row_v.at[pl.ds(0, ch)])
            @pl.when(c == 0)
            def _():
                pltpu.sync_copy(row_v.at[pl.ds(0, ch)], p0.at[sl])
                pltpu.sync_copy(cnt_sh.at[sl], one_v.at[pl.ds(0, ch)])
                pltpu.sync_copy(one_v.at[pl.ds(0, ch)], cnt.at[sl])
            @pl.when(c == 1)
            def _():
                pltpu.sync_copy(row_v.at[pl.ds(0, ch)], p1.at[sl])
            return carry
        lax.fori_loop(0, nq_s, fchunk, 0)

    f = pl.kernel(
        body,
        out_type=(jax.ShapeDtypeStruct((G, DH), jnp.float32),
                  jax.ShapeDtypeStruct((G, DH), jnp.float32),
                  jax.ShapeDtypeStruct((G, DH), jnp.float32)),
        mesh=mesh,
        scratch_types=[
            pltpu.VMEM((W,), jnp.int32),
            pltpu.VMEM((W, DH), jnp.float32),
            pltpu.VMEM((W, DH), jnp.float32),
            pltpu.VMEM_SHARED((G, DH), jnp.float32),
            pltpu.VMEM_SHARED((G, DH), jnp.float32),
        ],
        compiler_params=pltpu.CompilerParams(use_tc_tiling_on_sc=False),
        interpret=interpret,
    )
    return f(h0t, h1t, batch)


def _embed_tc(x, node_w, node_b, *, interpret=False):
    N, F = x.shape
    D = node_w.shape[1]
    B = _chunk(N, 10000)

    def body(x_ref, w_ref, b_ref, o_ref):
        h = jnp.dot(x_ref[...], w_ref[...],
                    preferred_element_type=jnp.float32) + b_ref[...]
        o_ref[...] = jnp.concatenate(
            [h, jnp.zeros((B, 128 - D), jnp.float32)], axis=1)

    return pl.pallas_call(
        body,
        grid=(N // B,),
        in_specs=[pl.BlockSpec((B, F), lambda i: (i, 0)),
                  pl.BlockSpec((F, D), lambda i: (0, 0)),
                  pl.BlockSpec((1, D), lambda i: (0, 0))],
        out_specs=pl.BlockSpec((B, 128), lambda i: (i, 0)),
        out_shape=jax.ShapeDtypeStruct((N, 128), jnp.float32),
        interpret=interpret,
    )(x, node_w, node_b.reshape(1, D))


def _mlp_stats_tc(h, a0, a1, w1, b1, w2, b2, *, interpret=False):
    N = h.shape[0]
    D = 2 * DH
    H = w1.shape[1]
    B = _chunk(N, 10000)

    def body(h_ref, a0_ref, a1_ref, w1_ref, b1_ref, w2_ref, b2_ref,
             z_ref, s_ref, q_ref):
        i = pl.program_id(0)
        xx = h_ref[:, :D] + jnp.concatenate([a0_ref[...], a1_ref[...]], axis=1)
        t = jnp.maximum(jnp.dot(xx, w1_ref[...],
                                preferred_element_type=jnp.float32)
                        + b1_ref[...], 0.0)
        z = jnp.dot(t, w2_ref[...], preferred_element_type=jnp.float32) \
            + b2_ref[...]
        z_ref[...] = z

        @pl.when(i == 0)
        def _():
            s_ref[...] = jnp.zeros_like(s_ref)
            q_ref[...] = jnp.zeros_like(q_ref)
        s_ref[...] += jnp.sum(z, axis=0, keepdims=True)
        q_ref[...] += jnp.sum(z * z, axis=0, keepdims=True)

    return pl.pallas_call(
        body,
        grid=(N // B,),
        in_specs=[pl.BlockSpec((B, 128), lambda i: (i, 0)),
                  pl.BlockSpec((B, DH), lambda i: (i, 0)),
                  pl.BlockSpec((B, DH), lambda i: (i, 0)),
                  pl.BlockSpec((D, H), lambda i: (0, 0)),
                  pl.BlockSpec((1, H), lambda i: (0, 0)),
                  pl.BlockSpec((H, D), lambda i: (0, 0)),
                  pl.BlockSpec((1, D), lambda i: (0, 0))],
        out_specs=(pl.BlockSpec((B, D), lambda i: (i, 0)),
                   pl.BlockSpec((1, D), lambda i: (0, 0)),
                   pl.BlockSpec((1, D), lambda i: (0, 0))),
        out_shape=(jax.ShapeDtypeStruct((N, D), jnp.float32),
                   jax.ShapeDtypeStruct((1, D), jnp.float32),
                   jax.ShapeDtypeStruct((1, D), jnp.float32)),
        interpret=interpret,
    )(h, a0, a1, w1, b1.reshape(1, H), w2, b2.reshape(1, D))


def _bn_relu_tc(z, zsum, zsq, g, bt, n_rows, *, interpret=False):
    N, D = z.shape
    B = _chunk(N, 10000)

    def body(z_ref, s_ref, q_ref, g_ref, b_ref, o_ref):
        inv_n = jnp.float32(1.0 / n_rows)
        mean = s_ref[...] * inv_n
        var = q_ref[...] * inv_n - mean * mean
        scale = g_ref[...] * lax.rsqrt(var + EPS_BN_)
        shift = b_ref[...] - mean * scale
        h = jnp.maximum(z_ref[...] * scale + shift, 0.0)
        o_ref[...] = jnp.concatenate(
            [h, jnp.zeros((B, 128 - D), jnp.float32)], axis=1)

    return pl.pallas_call(
        body,
        grid=(N // B,),
        in_specs=[pl.BlockSpec((B, D), lambda i: (i, 0)),
                  pl.BlockSpec((1, D), lambda i: (0, 0)),
                  pl.BlockSpec((1, D), lambda i: (0, 0)),
                  pl.BlockSpec((1, D), lambda i: (0, 0)),
                  pl.BlockSpec((1, D), lambda i: (0, 0))],
        out_specs=pl.BlockSpec((B, 128), lambda i: (i, 0)),
        out_shape=jax.ShapeDtypeStruct((N, 128), jnp.float32),
        interpret=interpret,
    )(z, zsum, zsq, g.reshape(1, D), bt.reshape(1, D))


def _head_tc(p0, p1, cnt, l1w, l1b, l2w, l2b, *, interpret=False):
    G = p0.shape[0]
    D = 2 * DH
    H = l1w.shape[1]
    O = l2w.shape[1]

    def body(p0_ref, p1_ref, c_ref, w1_ref, b1_ref, w2_ref, b2_ref, o_ref):
        ssum = jnp.concatenate([p0_ref[...], p1_ref[...]], axis=1)
        c = jnp.maximum(c_ref[...][:, 0:1], 1.0)
        gx = ssum / c
        t = jnp.maximum(jnp.dot(gx, w1_ref[...],
                                preferred_element_type=jnp.float32)
                        + b1_ref[...], 0.0)
        o_ref[...] = jnp.dot(t, w2_ref[...],
                             preferred_element_type=jnp.float32) + b2_ref[...]

    return pl.pallas_call(
        body,
        out_shape=jax.ShapeDtypeStruct((G, O), jnp.float32),
        interpret=interpret,
    )(p0, p1, cnt, l1w, l1b.reshape(1, H), l2w, l2b.reshape(1, O))


def _forward(x, edge_index, edge_attr, batch,
             node_w, node_b, edge_w, edge_b,
             conv0_w1, conv0_b1, conv0_w2, conv0_b2, bn0_g, bn0_b,
             conv1_w1, conv1_b1, conv1_w2, conv1_b2, bn1_g, bn1_b,
             lin1_w, lin1_b, lin2_w, lin2_b, G, interpret=False):
    N = x.shape[0]
    src = edge_index[0]
    dst = edge_index[1]
    ew2 = edge_w.reshape(3, 2, DH).transpose(1, 0, 2).reshape(6 * DH)
    eb2 = edge_b.reshape(2 * DH)
    attr_flat = edge_attr.reshape(-1)

    h128 = _embed_tc(x, node_w, node_b, interpret=interpret)
    layers = [
        (conv0_w1, conv0_b1, conv0_w2, conv0_b2, bn0_g, bn0_b),
        (conv1_w1, conv1_b1, conv1_w2, conv1_b2, bn1_g, bn1_b),
    ]
    for (w1, b1, w2, b2, g, bt) in layers:
        h0t, h1t = _split_sc(h128.reshape(-1), N, interpret=interpret)
        a0, a1 = _edge_phase(h0t, h1t, src, dst, attr_flat, ew2, eb2,
                             interpret=interpret)
        z, zsum, zsq = _mlp_stats_tc(h128, a0, a1, w1, b1, w2, b2,
                                     interpret=interpret)
        h128 = _bn_relu_tc(z, zsum, zsq, g, bt, N, interpret=interpret)

    h0t, h1t = _split_sc(h128.reshape(-1), N, interpret=interpret)
    p0, p1, cnt = _pool_phase(h0t, h1t, batch, G, interpret=interpret)
    return _head_tc(p0, p1, cnt, lin1_w, lin1_b, lin2_w, lin2_b,
                    interpret=interpret)


def kernel(x, edge_index, edge_attr, batch,
           node_w, node_b, edge_w, edge_b,
           conv0_w1, conv0_b1, conv0_w2, conv0_b2, bn0_g, bn0_b,
           conv1_w1, conv1_b1, conv1_w2, conv1_b2, bn1_g, bn1_b,
           lin1_w, lin1_b, lin2_w, lin2_b):
    return _forward(x, edge_index, edge_attr, batch,
                    node_w, node_b, edge_w, edge_b,
                    conv0_w1, conv0_b1, conv0_w2, conv0_b2, bn0_g, bn0_b,
                    conv1_w1, conv1_b1, conv1_w2, conv1_b2, bn1_g, bn1_b,
                    lin1_w, lin1_b, lin2_w, lin2_b, G=2000)


# speedup vs baseline: 1.1718x; 1.1310x over previous
import functools

import jax
import jax.numpy as jnp
from jax import lax
from jax.experimental import pallas as pl
from jax.experimental.pallas import tpu as pltpu
from jax.experimental.pallas import tpu_sc as plsc

NC = 2
NS = 16
DH = 16
EPS_BN_ = 1e-5


def _chunk(rows, cap):
    ch = min(rows, cap)
    while rows % ch:
        ch -= 1
    return ch


def _chunk8(total, cap):
    cand = [d for d in range(8, cap + 1, 8) if total % d == 0]
    assert cand, (total, cap)
    good = [d for d in cand if total // d >= NS]
    return max(good) if good else min(cand)


def _edge_phase(h0t, h1t, src, dst, attr_flat, ew2, eb2, *, interpret=False):
    N = h0t.shape[0]
    E = src.shape[0]
    ew = E // NS
    W = _chunk(ew, 1000)
    nwin = ew // W
    ch = _chunk8(N, min(1000, W))
    nq = N // ch

    mesh = plsc.VectorSubcoreMesh(core_axis_name="c", subcore_axis_name="s",
                                  num_cores=NC, num_subcores=NS)

    def body(h0_hbm, h1_hbm, src_hbm, dst_hbm, attr_hbm, ew_hbm, eb_hbm,
             out0, out1, ew_v, eb_v, idx_s, idx_d, att_v, row_v, aggr_sh):
        c = lax.axis_index("c")
        s = lax.axis_index("s")

        zero16 = jnp.zeros((DH,), jnp.float32)

        def zrow(i, carry):
            row_v[i] = zero16
            return carry
        lax.fori_loop(0, ch, zrow, 0)
        nq_s = lax.div(jnp.int32(nq) - s + jnp.int32(NS) - 1, jnp.int32(NS))

        def zchunk(k, carry):
            off = pl.multiple_of((s + k * NS) * ch, 8)
            pltpu.sync_copy(row_v.at[pl.ds(0, ch)],
                            aggr_sh.at[pl.ds(off, ch)])
            return carry
        lax.fori_loop(0, nq_s, zchunk, 0)
        plsc.subcore_barrier()

        for cc in range(NC):
            @pl.when(c == cc)
            def _():
                pltpu.sync_copy(ew_hbm.at[pl.ds(cc * 3 * DH, 3 * DH)], ew_v)
                pltpu.sync_copy(eb_hbm.at[pl.ds(cc * DH, DH)], eb_v)
        w0 = ew_v[pl.ds(0, DH)]
        w1 = ew_v[pl.ds(DH, DH)]
        w2 = ew_v[pl.ds(2 * DH, DH)]
        bb = eb_v[...]

        base0 = s * ew

        def window(g, carry):
            base = pl.multiple_of(base0 + g * W, 8)
            base3 = pl.multiple_of(base * 3, 8)
            pltpu.sync_copy(src_hbm.at[pl.ds(base, W)], idx_s)
            pltpu.sync_copy(dst_hbm.at[pl.ds(base, W)], idx_d)
            pltpu.sync_copy(attr_hbm.at[pl.ds(base3, 3 * W)],
                            att_v.at[pl.ds(0, 3 * W)])
            @pl.when(c == 0)
            def _():
                pltpu.sync_copy(h0_hbm.at[idx_s], row_v)
            @pl.when(c == 1)
            def _():
                pltpu.sync_copy(h1_hbm.at[idx_s], row_v)

            def edge(j, icarry):
                av = att_v[pl.ds(j * 3, DH)]
                hv = row_v[j]
                m = jnp.maximum(
                    hv + bb + av[0] * w0 + av[1] * w1 + av[2] * w2, 0.0)
                row_v[j] = m
                return icarry
            lax.fori_loop(0, W, edge, 0, unroll=4)
            pltpu.sync_copy(row_v, aggr_sh.at[idx_d], add=True)
            return carry
        lax.fori_loop(0, nwin, window, 0)
        plsc.subcore_barrier()

        def fchunk(k, carry):
            off = pl.multiple_of((s + k * NS) * ch, 8)
            sl = pl.ds(off, ch)
            pltpu.sync_copy(aggr_sh.at[sl], row_v.at[pl.ds(0, ch)])
            @pl.when(c == 0)
            def _():
                pltpu.sync_copy(row_v.at[pl.ds(0, ch)], out0.at[sl])
            @pl.when(c == 1)
            def _():
                pltpu.sync_copy(row_v.at[pl.ds(0, ch)], out1.at[sl])
            return carry
        lax.fori_loop(0, nq_s, fchunk, 0)

    f = pl.kernel(
        body,
        out_type=(jax.ShapeDtypeStruct((N, DH), jnp.float32),
                  jax.ShapeDtypeStruct((N, DH), jnp.float32)),
        mesh=mesh,
        scratch_types=[
            pltpu.VMEM((3 * DH,), jnp.float32),
            pltpu.VMEM((DH,), jnp.float32),
            pltpu.VMEM((W,), jnp.int32),
            pltpu.VMEM((W,), jnp.int32),
            pltpu.VMEM((3 * W + DH,), jnp.float32),
            pltpu.VMEM((W, DH), jnp.float32),
            pltpu.VMEM_SHARED((N, DH), jnp.float32),
        ],
        compiler_params=pltpu.CompilerParams(use_tc_tiling_on_sc=False),
        interpret=interpret,
    )
    return f(h0t, h1t, src, dst, attr_flat, ew2, eb2)


def _pool_phase(h0t, h1t, batch, G, *, interpret=False):
    N = h0t.shape[0]
    W = _chunk(N, 1000)
    nwin_total = N // W
    ch = _chunk8(G, min(1000, W))
    nq = G // ch

    mesh = plsc.VectorSubcoreMesh(core_axis_name="c", subcore_axis_name="s",
                                  num_cores=NC, num_subcores=NS)

    def body(h0_hbm, h1_hbm, b_hbm, p0, p1, cnt, idx_b, row_v, one_v,
             psum_sh, cnt_sh):
        c = lax.axis_index("c")
        s = lax.axis_index("s")

        zero16 = jnp.zeros((DH,), jnp.float32)
        one16 = jnp.ones((DH,), jnp.float32)

        def fill(i, carry):
            row_v[i] = zero16
            one_v[i] = one16
            return carry
        lax.fori_loop(0, W, fill, 0)
        nq_s = lax.div(jnp.int32(nq) - s + jnp.int32(NS) - 1, jnp.int32(NS))

        def zchunk(k, carry):
            sl = pl.ds(pl.multiple_of((s + k * NS) * ch, 8), ch)
            pltpu.sync_copy(row_v.at[pl.ds(0, ch)], psum_sh.at[sl])
            @pl.when(c == 0)
            def _():
                pltpu.sync_copy(row_v.at[pl.ds(0, ch)], cnt_sh.at[sl])
            return carry
        lax.fori_loop(0, nq_s, zchunk, 0)
        plsc.subcore_barrier()

        nw = lax.div(jnp.int32(nwin_total) - s + jnp.int32(NS) - 1, jnp.int32(NS))

        def window(k, carry):
            wid = s + k * NS
            base = pl.multiple_of(wid * W, 8)
            pltpu.sync_copy(b_hbm.at[pl.ds(base, W)], idx_b)
            @pl.when(c == 0)
            def _():
                pltpu.sync_copy(h0_hbm.at[pl.ds(base, W)], row_v)
            @pl.when(c == 1)
            def _():
                pltpu.sync_copy(h1_hbm.at[pl.ds(base, W)], row_v)
            pltpu.sync_copy(row_v, psum_sh.at[idx_b], add=True)
            @pl.when(c == 0)
            def _():
                pltpu.sync_copy(one_v, cnt_sh.at[idx_b], add=True)
            return carry
        lax.fori_loop(0, nw, window, 0)
        plsc.subcore_barrier()

        def fchunk(k, carry):
            sl = pl.ds(pl.multiple_of((s + k * NS) * ch, 8), ch)
            pltpu.sync_copy(psum_sh.at[sl], row_v.at[pl.ds(0, ch)])
            @pl.when(c == 0)
            def _():
                pltpu.sync_copy(row_v.at[pl.ds(0, ch)], p0.at[sl])
                pltpu.sync_copy(cnt_sh.at[sl], one_v.at[pl.ds(0, ch)])
                pltpu.sync_copy(one_v.at[pl.ds(0, ch)], cnt.at[sl])
            @pl.when(c == 1)
            def _():
                pltpu.sync_copy(row_v.at[pl.ds(0, ch)], p1.at[sl])
            return carry
        lax.fori_loop(0, nq_s, fchunk, 0)

    f = pl.kernel(
        body,
        out_type=(jax.ShapeDtypeStruct((G, DH), jnp.float32),
                  jax.ShapeDtypeStruct((G, DH), jnp.float32),
                  jax.ShapeDtypeStruct((G, DH), jnp.float32)),
        mesh=mesh,
        scratch_types=[
            pltpu.VMEM((W,), jnp.int32),
            pltpu.VMEM((W, DH), jnp.float32),
            pltpu.VMEM((W, DH), jnp.float32),
            pltpu.VMEM_SHARED((G, DH), jnp.float32),
            pltpu.VMEM_SHARED((G, DH), jnp.float32),
        ],
        compiler_params=pltpu.CompilerParams(use_tc_tiling_on_sc=False),
        interpret=interpret,
    )
    return f(h0t, h1t, batch)


def _embed_tc(x, node_w, node_b, *, interpret=False):
    N, F = x.shape
    D = node_w.shape[1]
    B = _chunk(N, 10000)

    def body(x_ref, w_ref, b_ref, o_ref, o0_ref, o1_ref):
        h = jnp.dot(x_ref[...], w_ref[...],
                    preferred_element_type=jnp.float32) + b_ref[...]
        o_ref[...] = h
        o0_ref[...] = h[:, :DH]
        o1_ref[...] = h[:, DH:]

    return pl.pallas_call(
        body,
        grid=(N // B,),
        in_specs=[pl.BlockSpec((B, F), lambda i: (i, 0)),
                  pl.BlockSpec((F, D), lambda i: (0, 0)),
                  pl.BlockSpec((1, D), lambda i: (0, 0))],
        out_specs=(pl.BlockSpec((B, D), lambda i: (i, 0)),
                   pl.BlockSpec((B, DH), lambda i: (i, 0)),
                   pl.BlockSpec((B, DH), lambda i: (i, 0))),
        out_shape=(jax.ShapeDtypeStruct((N, D), jnp.float32),
                   jax.ShapeDtypeStruct((N, DH), jnp.float32),
                   jax.ShapeDtypeStruct((N, DH), jnp.float32)),
        interpret=interpret,
    )(x, node_w, node_b.reshape(1, D))


def _mlp_stats_tc(h, a0, a1, w1, b1, w2, b2, *, interpret=False):
    N, D = h.shape
    H = w1.shape[1]
    B = _chunk(N, 10000)

    def body(h_ref, a0_ref, a1_ref, w1_ref, b1_ref, w2_ref, b2_ref,
             z_ref, s_ref, q_ref):
        i = pl.program_id(0)
        xx = h_ref[...] + jnp.concatenate([a0_ref[...], a1_ref[...]], axis=1)
        t = jnp.maximum(jnp.dot(xx, w1_ref[...],
                                preferred_element_type=jnp.float32)
                        + b1_ref[...], 0.0)
        z = jnp.dot(t, w2_ref[...], preferred_element_type=jnp.float32) \
            + b2_ref[...]
        z_ref[...] = z

        @pl.when(i == 0)
        def _():
            s_ref[...] = jnp.zeros_like(s_ref)
            q_ref[...] = jnp.zeros_like(q_ref)
        s_ref[...] += jnp.sum(z, axis=0, keepdims=True)
        q_ref[...] += jnp.sum(z * z, axis=0, keepdims=True)

    return pl.pallas_call(
        body,
        grid=(N // B,),
        in_specs=[pl.BlockSpec((B, D), lambda i: (i, 0)),
                  pl.BlockSpec((B, DH), lambda i: (i, 0)),
                  pl.BlockSpec((B, DH), lambda i: (i, 0)),
                  pl.BlockSpec((D, H), lambda i: (0, 0)),
                  pl.BlockSpec((1, H), lambda i: (0, 0)),
                  pl.BlockSpec((H, D), lambda i: (0, 0)),
                  pl.BlockSpec((1, D), lambda i: (0, 0))],
        out_specs=(pl.BlockSpec((B, D), lambda i: (i, 0)),
                   pl.BlockSpec((1, D), lambda i: (0, 0)),
                   pl.BlockSpec((1, D), lambda i: (0, 0))),
        out_shape=(jax.ShapeDtypeStruct((N, D), jnp.float32),
                   jax.ShapeDtypeStruct((1, D), jnp.float32),
                   jax.ShapeDtypeStruct((1, D), jnp.float32)),
        interpret=interpret,
    )(h, a0, a1, w1, b1.reshape(1, H), w2, b2.reshape(1, D))


def _bnvar_tc(z, zsum, n_rows, *, interpret=False):
    N, D = z.shape
    B = _chunk(N, 10000)

    def body(z_ref, s_ref, v_ref):
        i = pl.program_id(0)
        mean = s_ref[...] * jnp.float32(1.0 / n_rows)
        d = z_ref[...] - mean

        @pl.when(i == 0)
        def _():
            v_ref[...] = jnp.zeros_like(v_ref)
        v_ref[...] += jnp.sum(d * d, axis=0, keepdims=True)

    return pl.pallas_call(
        body,
        grid=(N // B,),
        in_specs=[pl.BlockSpec((B, D), lambda i: (i, 0)),
                  pl.BlockSpec((1, D), lambda i: (0, 0))],
        out_specs=pl.BlockSpec((1, D), lambda i: (0, 0)),
        out_shape=jax.ShapeDtypeStruct((1, D), jnp.float32),
        interpret=interpret,
    )(z, zsum)


def _bn_relu_tc(z, zsum, varsum, g, bt, n_rows, *, interpret=False):
    N, D = z.shape
    B = _chunk(N, 10000)

    def body(z_ref, s_ref, q_ref, g_ref, b_ref, o_ref, o0_ref, o1_ref):
        inv_n = jnp.float32(1.0 / n_rows)
        mean = s_ref[...] * inv_n
        var = q_ref[...] * inv_n
        scale = g_ref[...] * lax.rsqrt(var + EPS_BN_)
        shift = b_ref[...] - mean * scale
        h = jnp.maximum(z_ref[...] * scale + shift, 0.0)
        o_ref[...] = h
        o0_ref[...] = h[:, :DH]
        o1_ref[...] = h[:, DH:]

    return pl.pallas_call(
        body,
        grid=(N // B,),
        in_specs=[pl.BlockSpec((B, D), lambda i: (i, 0)),
                  pl.BlockSpec((1, D), lambda i: (0, 0)),
                  pl.BlockSpec((1, D), lambda i: (0, 0)),
                  pl.BlockSpec((1, D), lambda i: (0, 0)),
                  pl.BlockSpec((1, D), lambda i: (0, 0))],
        out_specs=(pl.BlockSpec((B, D), lambda i: (i, 0)),
                   pl.BlockSpec((B, DH), lambda i: (i, 0)),
                   pl.BlockSpec((B, DH), lambda i: (i, 0))),
        out_shape=(jax.ShapeDtypeStruct((N, D), jnp.float32),
                   jax.ShapeDtypeStruct((N, DH), jnp.float32),
                   jax.ShapeDtypeStruct((N, DH), jnp.float32)),
        interpret=interpret,
    )(z, zsum, varsum, g.reshape(1, D), bt.reshape(1, D))


def _head_tc(p0, p1, cnt, l1w, l1b, l2w, l2b, *, interpret=False):
    G = p0.shape[0]
    D = 2 * DH
    H = l1w.shape[1]
    O = l2w.shape[1]

    def body(p0_ref, p1_ref, c_ref, w1_ref, b1_ref, w2_ref, b2_ref, o_ref):
        ssum = jnp.concatenate([p0_ref[...], p1_ref[...]], axis=1)
        c = jnp.maximum(c_ref[...][:, 0:1], 1.0)
        gx = ssum / c
        t = jnp.maximum(jnp.dot(gx, w1_ref[...],
                                preferred_element_type=jnp.float32)
                        + b1_ref[...], 0.0)
        o_ref[...] = jnp.dot(t, w2_ref[...],
                             preferred_element_type=jnp.float32) + b2_ref[...]

    return pl.pallas_call(
        body,
        out_shape=jax.ShapeDtypeStruct((G, O), jnp.float32),
        interpret=interpret,
    )(p0, p1, cnt, l1w, l1b.reshape(1, H), l2w, l2b.reshape(1, O))


def _forward(x, edge_index, edge_attr, batch,
             node_w, node_b, edge_w, edge_b,
             conv0_w1, conv0_b1, conv0_w2, conv0_b2, bn0_g, bn0_b,
             conv1_w1, conv1_b1, conv1_w2, conv1_b2, bn1_g, bn1_b,
             lin1_w, lin1_b, lin2_w, lin2_b, G, interpret=False):
    N = x.shape[0]
    src = edge_index[0]
    dst = edge_index[1]
    ew2 = edge_w.reshape(3, 2, DH).transpose(1, 0, 2).reshape(6 * DH)
    eb2 = edge_b.reshape(2 * DH)
    attr_flat = edge_attr.reshape(-1)

    h, _h0d, _h1d = _embed_tc(x, node_w, node_b, interpret=interpret)
    h0t = lax.slice(h, (0, 0), (N, DH))
    h1t = lax.slice(h, (0, DH), (N, 2 * DH))
    layers = [
        (conv0_w1, conv0_b1, conv0_w2, conv0_b2, bn0_g, bn0_b),
        (conv1_w1, conv1_b1, conv1_w2, conv1_b2, bn1_g, bn1_b),
    ]
    for (w1, b1, w2, b2, g, bt) in layers:
        a0, a1 = _edge_phase(h0t, h1t, src, dst, attr_flat, ew2, eb2,
                             interpret=interpret)
        z, zsum, _zsq = _mlp_stats_tc(h, a0, a1, w1, b1, w2, b2,
                                      interpret=interpret)
        varsum = _bnvar_tc(z, zsum, N, interpret=interpret)
        h, _h0d, _h1d = _bn_relu_tc(z, zsum, varsum, g, bt, N,
                                    interpret=interpret)
        h0t = lax.slice(h, (0, 0), (N, DH))
        h1t = lax.slice(h, (0, DH), (N, 2 * DH))

    p0, p1, cnt = _pool_phase(h0t, h1t, batch, G, interpret=interpret)
    return _head_tc(p0, p1, cnt, lin1_w, lin1_b, lin2_w, lin2_b,
                    interpret=interpret)


def kernel(x, edge_index, edge_attr, batch,
           node_w, node_b, edge_w, edge_b,
           conv0_w1, conv0_b1, conv0_w2, conv0_b2, bn0_g, bn0_b,
           conv1_w1, conv1_b1, conv1_w2, conv1_b2, bn1_g, bn1_b,
           lin1_w, lin1_b, lin2_w, lin2_b):
    return _forward(x, edge_index, edge_attr, batch,
                    node_w, node_b, edge_w, edge_b,
                    conv0_w1, conv0_b1, conv0_w2, conv0_b2, bn0_g, bn0_b,
                    conv1_w1, conv1_b1, conv1_w2, conv1_b2, bn1_g, bn1_b,
                    lin1_w, lin1_b, lin2_w, lin2_b, G=2000)
